# baseline (speedup 1.0000x reference)
import functools

import jax
import jax.numpy as jnp
from jax import lax
from jax.experimental import pallas as pl
from jax.experimental.pallas import tpu as pltpu
from jax.experimental.pallas import tpu_sc as plsc

B = 4096
L = 50
H = 64
V = 100000
LANES = 16
HV = H // LANES

NC, NS = 2, 16
NW = NC * NS
BPW = B // NW

CB = 16
STEPS = BPW // CB
NB = 8
GCH = 32
NG = (CB * L) // GCH

VCH = 256
NFULL = V // VCH
VTAIL = V - NFULL * VCH
KMAX = (NFULL + NW - 1) // NW


def _repack_body(tt_hbm, tail_hbm, out_hbm, in0, in1, outv, sem0, sem1):
    wid = lax.axis_index("c") * NS + lax.axis_index("s")
    in_bufs = (in0, in1)
    sems = (sem0, sem1)

    rows_h = [h * LANES + lax.iota(jnp.int32, LANES) for h in range(HV)]

    def transpose_chunk(in_v, nrows):
        def body(q4, carry):
            q0 = q4 * 4
            for dq in range(4):
                for hi in range(2):
                    col = jnp.full((LANES,), 2 * (q0 + dq) + hi, jnp.int32)
                    for h in range(HV):
                        vec = plsc.load_gather(in_v, [rows_h[h], col])
                        outv[q0 + dq, pl.ds(hi * H + h * LANES, LANES)] = vec
            return carry
        lax.fori_loop(0, nrows // 4, body, 0)

    def fire(k, c):
        buf = in_bufs[k % 2]
        return pltpu.async_copy(tt_hbm.at[:, pl.ds(c * VCH, VCH)], buf,
                                sems[k % 2])

    wc0 = wid
    d = fire(0, wc0)
    for k in range(KMAX):
        c = wid + k * NW
        nxt = wid + (k + 1) * NW
        if k + 1 < KMAX:
            @pl.when(nxt < NFULL)
            def _():
                fire(k + 1, nxt)
        @pl.when(c < NFULL)
        def _():
            pltpu.make_async_copy(
                tt_hbm.at[:, pl.ds(c * VCH, VCH)], in_bufs[k % 2],
                sems[k % 2]).wait()
            transpose_chunk(in_bufs[k % 2], VCH // 2)
            pltpu.sync_copy(outv, out_hbm.at[pl.ds(c * (VCH // 2), VCH // 2)])

    @pl.when(wid == NW - 1)
    def _():
        pltpu.sync_copy(tail_hbm, in0)
        transpose_chunk(in0, VTAIL // 2)
        pltpu.sync_copy(outv.at[pl.ds(0, VTAIL // 2)],
                        out_hbm.at[pl.ds(NFULL * VCH // 2, VTAIL // 2)])


def _bow_body(table_hbm, idx_hbm, w_hbm, out_hbm,
              idx_v, rows0, rows1, w_tmp, w_v, out_v, sem0, sem1):
    wid = lax.axis_index("c") * NS + lax.axis_index("s")
    row_base = wid * BPW

    pltpu.sync_copy(idx_hbm.at[pl.ds(wid * 50, 50)], idx_v)
    pltpu.sync_copy(w_hbm, w_tmp)
    for l in range(L):
        for h in range(HV):
            flat = l * H + h * LANES
            w_v[l, pl.ds(h * LANES, LANES)] = \
                w_tmp[flat // 128, pl.ds(flat % 128, LANES)]

    rows_bufs = (rows0, rows1)
    sems = (sem0, sem1)

    def fire(g):
        buf = rows_bufs[g % 2]
        sem = sems[g % 2]
        descs = []
        for j in range(NG):
            gc = g * NG + j
            src_idx = idx_v.at[gc // 4, pl.ds((gc % 4) * GCH, GCH)]
            descs.append(pltpu.async_copy(
                table_hbm.at[src_idx],
                buf.at[pl.ds(j * GCH, GCH)],
                sem))
        return descs

    pending = {0: fire(0)}

    for g in range(STEPS):
        if g + 1 < STEPS:
            pending[g + 1] = fire(g + 1)
        for d in pending.pop(g):
            d.wait()
        rows = rows_bufs[g % 2]

        for bb in range(CB // NB):
            def step(l, accs, rows=rows, bb=bb):
                out = []
                ws = [w_v[l, pl.ds(h * LANES, LANES)] for h in range(HV)]
                for r in range(NB):
                    ridx = (bb * NB + r) * L + l
                    for h in range(HV):
                        out.append(accs[r * HV + h]
                                   + rows[ridx, pl.ds(h * LANES, LANES)] * ws[h])
                return tuple(out)

            zero = jnp.zeros((LANES,), jnp.float32)
            accs = lax.fori_loop(0, L, step, (zero,) * (NB * HV))
            for r in range(NB):
                for h in range(HV):
                    out_v[bb * NB + r, pl.ds(h * LANES, LANES)] = accs[r * HV + h]

        pltpu.sync_copy(out_v, out_hbm.at[pl.ds(row_base + g * CB, CB)])


def _mesh():
    return plsc.VectorSubcoreMesh(core_axis_name="c", subcore_axis_name="s",
                                  num_cores=NC, num_subcores=NS)


@jax.jit
def _bow(table, idx, w):
    t2 = pl.kernel(
        _repack_body,
        out_type=jax.ShapeDtypeStruct((V // 2, 128), jnp.float32),
        mesh=_mesh(),
        compiler_params=pltpu.CompilerParams(use_tc_tiling_on_sc=True,
                                             needs_layout_passes=False),
        scratch_types=[
            pltpu.VMEM((H, VCH), jnp.float32),
            pltpu.VMEM((H, VCH), jnp.float32),
            pltpu.VMEM((VCH // 2, 128), jnp.float32),
            pltpu.SemaphoreType.DMA,
            pltpu.SemaphoreType.DMA,
        ],
    )(table.T, jnp.pad(table[NFULL * VCH:], ((0, VCH - VTAIL), (0, 0))).T)
    t3 = t2.reshape(V, H)
    return pl.kernel(
        _bow_body,
        out_type=jax.ShapeDtypeStruct((B, H), jnp.float32),
        mesh=_mesh(),
        compiler_params=pltpu.CompilerParams(use_tc_tiling_on_sc=False),
        scratch_types=[
            pltpu.VMEM((BPW * L // 128, 128), jnp.int32),
            pltpu.VMEM((CB * L, H), jnp.float32),
            pltpu.VMEM((CB * L, H), jnp.float32),
            pltpu.VMEM((L * H // 128, 128), jnp.float32),
            pltpu.VMEM((L, H), jnp.float32),
            pltpu.VMEM((CB, H), jnp.float32),
            pltpu.SemaphoreType.DMA,
            pltpu.SemaphoreType.DMA,
        ],
    )(t3, idx, w)


def kernel(input, table, weights):
    idx = input.reshape(B * L // 128, 128)
    w = weights[:L].reshape(L * H // 128, 128)
    return _bow(table, idx, w)


# speedup vs baseline: 1.1978x; 1.1978x over previous
import functools

import jax
import jax.numpy as jnp
from jax import lax
from jax.experimental import pallas as pl
from jax.experimental.pallas import tpu as pltpu
from jax.experimental.pallas import tpu_sc as plsc

B = 4096
L = 50
H = 64
HH = H // 2
LANES = 16
HV = HH // LANES

NC, NS = 2, 16
NW = NC * NS
BPW = B // NW

CB = 16
STEPS = BPW // CB
NB = 8
GCH = 32
NG = (CB * L) // GCH


def _bow_body(woff, table_hbm, idx_hbm, w_hbm, out_hbm,
              idx_v, rows0, rows1, w_tmp, w_v, out_v, sem0, sem1):
    wid = lax.axis_index("c") * NS + lax.axis_index("s")
    row_base = wid * BPW

    pltpu.sync_copy(idx_hbm.at[pl.ds(wid * 50, 50)], idx_v)
    pltpu.sync_copy(w_hbm, w_tmp)
    for l in range(L):
        for h in range(HV):
            flat = l * H + woff + h * LANES
            w_v[l, pl.ds(h * LANES, LANES)] = \
                w_tmp[flat // 128, pl.ds(flat % 128, LANES)]

    rows_bufs = (rows0, rows1)
    sems = (sem0, sem1)

    def fire(g):
        buf = rows_bufs[g % 2]
        sem = sems[g % 2]
        descs = []
        for j in range(NG):
            gc = g * NG + j
            src_idx = idx_v.at[gc // 4, pl.ds((gc % 4) * GCH, GCH)]
            descs.append(pltpu.async_copy(
                table_hbm.at[src_idx],
                buf.at[pl.ds(j * GCH, GCH)],
                sem))
        return descs

    pending = {0: fire(0)}

    for g in range(STEPS):
        if g + 1 < STEPS:
            pending[g + 1] = fire(g + 1)
        for d in pending.pop(g):
            d.wait()
        rows = rows_bufs[g % 2]

        for bb in range(CB // NB):
            def step(l, accs, rows=rows, bb=bb):
                out = []
                ws = [w_v[l, pl.ds(h * LANES, LANES)] for h in range(HV)]
                for r in range(NB):
                    ridx = (bb * NB + r) * L + l
                    for h in range(HV):
                        out.append(accs[r * HV + h]
                                   + rows[ridx, pl.ds(h * LANES, LANES)] * ws[h])
                return tuple(out)

            zero = jnp.zeros((LANES,), jnp.float32)
            accs = lax.fori_loop(0, L, step, (zero,) * (NB * HV))
            for r in range(NB):
                for h in range(HV):
                    out_v[bb * NB + r, pl.ds(h * LANES, LANES)] = accs[r * HV + h]

        pltpu.sync_copy(out_v, out_hbm.at[pl.ds(row_base + g * CB, CB)])


def _half_call(woff):
    mesh = plsc.VectorSubcoreMesh(core_axis_name="c", subcore_axis_name="s",
                                  num_cores=NC, num_subcores=NS)
    return pl.kernel(
        functools.partial(_bow_body, woff),
        out_type=jax.ShapeDtypeStruct((B, HH), jnp.float32),
        mesh=mesh,
        compiler_params=pltpu.CompilerParams(use_tc_tiling_on_sc=False),
        scratch_types=[
            pltpu.VMEM((BPW * L // 128, 128), jnp.int32),
            pltpu.VMEM((CB * L, HH), jnp.float32),
            pltpu.VMEM((CB * L, HH), jnp.float32),
            pltpu.VMEM((L * H // 128, 128), jnp.float32),
            pltpu.VMEM((L, HH), jnp.float32),
            pltpu.VMEM((CB, HH), jnp.float32),
            pltpu.SemaphoreType.DMA,
            pltpu.SemaphoreType.DMA,
        ],
    )


@jax.jit
def _bow(table, idx, w):
    outs = [_half_call(woff)(table[:, woff:woff + HH], idx, w)
            for woff in (0, HH)]
    return jnp.concatenate(outs, axis=1)


def kernel(input, table, weights):
    idx = input.reshape(B * L // 128, 128)
    w = weights[:L].reshape(L * H // 128, 128)
    return _bow(table, idx, w)


# speedup vs baseline: 2.0521x; 1.7132x over previous
import functools

import jax
import jax.numpy as jnp
from jax import lax
from jax.experimental import pallas as pl
from jax.experimental.pallas import tpu as pltpu
from jax.experimental.pallas import tpu_sc as plsc

B = 4096
L = 50
H = 64
LANES = 16
HV = H // LANES

NC, NS = 2, 16
NW = NC * NS
BPW = B // NW

CB = 16
STEPS = BPW // CB
NB = 8
GCH = 32
NG = (CB * L) // GCH


def _bow_body(table_hbm, idx_hbm, w_hbm, out_hbm,
              idx_v, rows0, rows1, w_tmp, w_v, out_v, sem0, sem1):
    wid = lax.axis_index("c") * NS + lax.axis_index("s")
    row_base = wid * BPW

    pltpu.sync_copy(idx_hbm.at[pl.ds(wid * 50, 50)], idx_v)
    pltpu.sync_copy(w_hbm, w_tmp)
    for l in range(L):
        for h in range(HV):
            flat = l * H + h * LANES
            w_v[l, pl.ds(h * LANES, LANES)] = \
                w_tmp[flat // 128, pl.ds(flat % 128, LANES)]

    rows_bufs = (rows0, rows1)
    sems = (sem0, sem1)

    def fire(g):
        buf = rows_bufs[g % 2]
        sem = sems[g % 2]
        descs = []
        for j in range(NG):
            gc = g * NG + j
            src_idx = idx_v.at[gc // 4, pl.ds((gc % 4) * GCH, GCH)]
            descs.append(pltpu.async_copy(
                table_hbm.at[src_idx],
                buf.at[pl.ds(j * GCH, GCH)],
                sem))
        return descs

    pending = {0: fire(0)}

    for g in range(STEPS):
        if g + 1 < STEPS:
            pending[g + 1] = fire(g + 1)
        for d in pending.pop(g):
            d.wait()
        rows = rows_bufs[g % 2]

        for bb in range(CB // NB):
            def step(l, accs, rows=rows, bb=bb):
                out = []
                ws = [w_v[l, pl.ds(h * LANES, LANES)] for h in range(HV)]
                for r in range(NB):
                    ridx = (bb * NB + r) * L + l
                    for h in range(HV):
                        out.append(accs[r * HV + h]
                                   + rows[ridx, pl.ds(h * LANES, LANES)] * ws[h])
                return tuple(out)

            zero = jnp.zeros((LANES,), jnp.float32)
            accs = lax.fori_loop(0, L, step, (zero,) * (NB * HV))
            for r in range(NB):
                for h in range(HV):
                    out_v[bb * NB + r, pl.ds(h * LANES, LANES)] = accs[r * HV + h]

        pltpu.sync_copy(out_v, out_hbm.at[pl.ds(row_base + g * CB, CB)])


@jax.jit
def _bow(table, idx, w):
    mesh = plsc.VectorSubcoreMesh(core_axis_name="c", subcore_axis_name="s",
                                  num_cores=NC, num_subcores=NS)
    return pl.kernel(
        _bow_body,
        out_type=jax.ShapeDtypeStruct((B, H), jnp.float32),
        mesh=mesh,
        compiler_params=pltpu.CompilerParams(use_tc_tiling_on_sc=False),
        scratch_types=[
            pltpu.VMEM((BPW * L // 128, 128), jnp.int32),
            pltpu.VMEM((CB * L, H), jnp.float32),
            pltpu.VMEM((CB * L, H), jnp.float32),
            pltpu.VMEM((L * H // 128, 128), jnp.float32),
            pltpu.VMEM((L, H), jnp.float32),
            pltpu.VMEM((CB, H), jnp.float32),
            pltpu.SemaphoreType.DMA,
            pltpu.SemaphoreType.DMA,
        ],
    )(table, idx, w)


def kernel(input, table, weights):
    idx = input.reshape(B * L // 128, 128)
    w = weights[:L].reshape(L * H // 128, 128)
    return _bow(table, idx, w)
